# baseline (speedup 1.0000x reference)
import functools
import math

import jax
import jax.numpy as jnp
import numpy as np
from jax import lax
from jax.experimental import pallas as pl
from jax.experimental.pallas import tpu as pltpu, tpu_sc as plsc

EMBED_DIM = 128
SEQ = 2048
BATCH = 4
SCALE = math.sqrt(EMBED_DIM)

NW = 32
ROWS = BATCH * SEQ
RPW = ROWS // NW
CHUNK = 64
CPW = RPW // CHUNK
WPB = SEQ // RPW
L = 16


def _pe_const():
    position_id = np.arange(0, SEQ, dtype=np.float32)[:, None]
    frequencies = np.power(
        10000.0, -np.arange(0, EMBED_DIM, 2, dtype=np.float32) / EMBED_DIM)
    sin_part = np.sin(position_id * frequencies)
    cos_part = np.cos(position_id * frequencies)
    pe = np.stack([sin_part, cos_part], axis=-1).reshape(SEQ, EMBED_DIM)
    bf = pe.astype(jnp.bfloat16)
    u16 = np.asarray(bf).view(np.uint16).reshape(SEQ, EMBED_DIM // 32, 2, L)
    words = u16[:, :, 0, :].astype(np.uint32) | (
        u16[:, :, 1, :].astype(np.uint32) << 16)
    return words.reshape(SEQ, EMBED_DIM // 2).view(np.int32)


_PE_PACKED = _pe_const()


@functools.partial(
    pl.kernel,
    mesh=plsc.VectorSubcoreMesh(core_axis_name="c", subcore_axis_name="s"),
    out_type=(
        jax.ShapeDtypeStruct((BATCH, SEQ, EMBED_DIM), jnp.float32),
        jax.ShapeDtypeStruct((BATCH, SEQ, EMBED_DIM), jnp.float32),
    ),
    scratch_types=[
        pltpu.VMEM((RPW,), jnp.int32),
        pltpu.VMEM((RPW,), jnp.int32),
        pltpu.VMEM((RPW, EMBED_DIM // 2), jnp.int32),
        pltpu.VMEM((RPW, EMBED_DIM), jnp.float32),
        pltpu.VMEM((RPW, EMBED_DIM), jnp.float32),
        pltpu.SemaphoreType.DMA,
        pltpu.SemaphoreType.DMA,
        pltpu.SemaphoreType.DMA,
        pltpu.SemaphoreType.DMA,
        pltpu.SemaphoreType.DMA,
        pltpu.SemaphoreType.DMA,
    ],
)
def _sc_embed(src_ids, tgt_ids, src_tab, tgt_tab, pe,
              src_out, tgt_out,
              idx_s, idx_t, pe_v, rows_s, rows_t,
              sem_s, sem_t, sem_p, sem_i, sem_os, sem_ot):
    cid = lax.axis_index("c")
    sid = lax.axis_index("s")
    wid = sid * 2 + cid
    b = wid // WPB
    s0 = lax.rem(wid, WPB) * RPW

    p0 = pl.multiple_of(s0, RPW)
    cp_pe = pltpu.async_copy(pe.at[pl.ds(p0, RPW)], pe_v, sem_p)
    cp_is = pltpu.async_copy(src_ids.at[b, pl.ds(s0, RPW)], idx_s, sem_i)
    cp_it = pltpu.async_copy(tgt_ids.at[b, pl.ds(s0, RPW)], idx_t, sem_i)
    cp_is.wait()
    cps = [pltpu.async_copy(src_tab.at[idx_s.at[pl.ds(j * CHUNK, CHUNK)]],
                            rows_s.at[pl.ds(j * CHUNK, CHUNK)], sem_s)
           for j in range(CPW)]
    cp_it.wait()
    cpt = [pltpu.async_copy(tgt_tab.at[idx_t.at[pl.ds(j * CHUNK, CHUNK)]],
                            rows_t.at[pl.ds(j * CHUNK, CHUNK)], sem_t)
           for j in range(CPW)]

    hi_mask = jnp.int32(-65536)

    def fma_chunk(rows_ref, j):
        def body(r, _):
            for h in range(EMBED_DIM // (2 * L)):
                pv = pe_v[r, pl.ds(h * L, L)]
                pa = lax.bitcast_convert_type(
                    lax.shift_left(pv, 16), jnp.float32)
                pb = lax.bitcast_convert_type(
                    lax.bitwise_and(pv, hi_mask), jnp.float32)
                sl0 = pl.ds(h * 2 * L, L)
                sl1 = pl.ds(h * 2 * L + L, L)
                rows_ref[r, sl0] = rows_ref[r, sl0] * SCALE + pa
                rows_ref[r, sl1] = rows_ref[r, sl1] * SCALE + pb
            return 0
        lax.fori_loop(j * CHUNK, (j + 1) * CHUNK, body, 0)

    cp_pe.wait()
    sts = []
    for j in range(CPW):
        cps[j].wait()
        fma_chunk(rows_s, j)
        sts.append(pltpu.async_copy(
            rows_s.at[pl.ds(j * CHUNK, CHUNK)],
            src_out.at[b, pl.ds(s0 + j * CHUNK, CHUNK)], sem_os))
    for j in range(CPW):
        cpt[j].wait()
        fma_chunk(rows_t, j)
        sts.append(pltpu.async_copy(
            rows_t.at[pl.ds(j * CHUNK, CHUNK)],
            tgt_out.at[b, pl.ds(s0 + j * CHUNK, CHUNK)], sem_ot))
    for st in sts:
        st.wait()


def kernel(src_token_ids_batch, tgt_token_ids_batch, src_table, tgt_table):
    pe = jnp.asarray(_PE_PACKED)
    src_ids = src_token_ids_batch.astype(jnp.int32)
    tgt_ids = tgt_token_ids_batch.astype(jnp.int32)
    return _sc_embed(src_ids, tgt_ids, src_table, tgt_table, pe)


# speedup vs baseline: 1.0162x; 1.0162x over previous
import functools
import math

import jax
import jax.numpy as jnp
import numpy as np
from jax import lax
from jax.experimental import pallas as pl
from jax.experimental.pallas import tpu as pltpu, tpu_sc as plsc

EMBED_DIM = 128
SEQ = 2048
BATCH = 4
SCALE = math.sqrt(EMBED_DIM)

NW = 32
ROWS = BATCH * SEQ
RPW = ROWS // NW
CHUNK = 128
CPW = RPW // CHUNK
WPB = SEQ // RPW
L = 16


def _pe_const():
    position_id = np.arange(0, SEQ, dtype=np.float32)[:, None]
    frequencies = np.power(
        10000.0, -np.arange(0, EMBED_DIM, 2, dtype=np.float32) / EMBED_DIM)
    sin_part = np.sin(position_id * frequencies)
    cos_part = np.cos(position_id * frequencies)
    pe = np.stack([sin_part, cos_part], axis=-1).reshape(SEQ, EMBED_DIM)
    bf = pe.astype(jnp.bfloat16)
    u16 = np.asarray(bf).view(np.uint16).reshape(SEQ, EMBED_DIM // 32, 2, L)
    words = u16[:, :, 0, :].astype(np.uint32) | (
        u16[:, :, 1, :].astype(np.uint32) << 16)
    return words.reshape(SEQ, EMBED_DIM // 2).view(np.int32)


_PE_PACKED = _pe_const()


@functools.partial(
    pl.kernel,
    mesh=plsc.VectorSubcoreMesh(core_axis_name="c", subcore_axis_name="s"),
    out_type=(
        jax.ShapeDtypeStruct((BATCH, SEQ, EMBED_DIM), jnp.float32),
        jax.ShapeDtypeStruct((BATCH, SEQ, EMBED_DIM), jnp.float32),
    ),
    scratch_types=[
        pltpu.VMEM((RPW,), jnp.int32),
        pltpu.VMEM((RPW,), jnp.int32),
        pltpu.VMEM((RPW, EMBED_DIM // 2), jnp.int32),
        pltpu.VMEM((RPW, EMBED_DIM), jnp.float32),
        pltpu.VMEM((RPW, EMBED_DIM), jnp.float32),
        pltpu.SemaphoreType.DMA,
        pltpu.SemaphoreType.DMA,
        pltpu.SemaphoreType.DMA,
        pltpu.SemaphoreType.DMA,
        pltpu.SemaphoreType.DMA,
        pltpu.SemaphoreType.DMA,
        pltpu.SemaphoreType.DMA,
    ],
)
def _sc_embed(src_ids, tgt_ids, src_tab, tgt_tab, pe,
              src_out, tgt_out,
              idx_s, idx_t, pe_v, rows_s, rows_t,
              sem_s, sem_t, sem_p, sem_i, sem_it, sem_os, sem_ot):
    cid = lax.axis_index("c")
    sid = lax.axis_index("s")
    wid = sid * 2 + cid
    b = wid // WPB
    s0 = lax.rem(wid, WPB) * RPW

    p0 = pl.multiple_of(s0, RPW)
    cp_is = pltpu.async_copy(src_ids.at[b, pl.ds(s0, RPW)], idx_s, sem_i)
    cp_it = pltpu.async_copy(tgt_ids.at[b, pl.ds(s0, RPW)], idx_t, sem_it)
    cp_pe = pltpu.async_copy(pe.at[pl.ds(p0, RPW)], pe_v, sem_p)
    cp_is.wait()
    cps = [pltpu.async_copy(src_tab.at[idx_s.at[pl.ds(j * CHUNK, CHUNK)]],
                            rows_s.at[pl.ds(j * CHUNK, CHUNK)], sem_s)
           for j in range(CPW)]
    cp_it.wait()
    cpt = [pltpu.async_copy(tgt_tab.at[idx_t.at[pl.ds(j * CHUNK, CHUNK)]],
                            rows_t.at[pl.ds(j * CHUNK, CHUNK)], sem_t)
           for j in range(CPW)]

    hi_mask = jnp.int32(-65536)

    def fma_chunk(rows_ref, j):
        def body(r, _):
            for h in range(EMBED_DIM // (2 * L)):
                pv = pe_v[r, pl.ds(h * L, L)]
                pa = lax.bitcast_convert_type(
                    lax.shift_left(pv, 16), jnp.float32)
                pb = lax.bitcast_convert_type(
                    lax.bitwise_and(pv, hi_mask), jnp.float32)
                sl0 = pl.ds(h * 2 * L, L)
                sl1 = pl.ds(h * 2 * L + L, L)
                rows_ref[r, sl0] = rows_ref[r, sl0] * SCALE + pa
                rows_ref[r, sl1] = rows_ref[r, sl1] * SCALE + pb
            return 0
        lax.fori_loop(j * CHUNK, (j + 1) * CHUNK, body, 0)

    cp_pe.wait()
    sts = []
    for j in range(CPW):
        cps[j].wait()
        fma_chunk(rows_s, j)
        sts.append(pltpu.async_copy(
            rows_s.at[pl.ds(j * CHUNK, CHUNK)],
            src_out.at[b, pl.ds(s0 + j * CHUNK, CHUNK)], sem_os))
    for j in range(CPW):
        cpt[j].wait()
        fma_chunk(rows_t, j)
        sts.append(pltpu.async_copy(
            rows_t.at[pl.ds(j * CHUNK, CHUNK)],
            tgt_out.at[b, pl.ds(s0 + j * CHUNK, CHUNK)], sem_ot))
    for st in sts:
        st.wait()


def kernel(src_token_ids_batch, tgt_token_ids_batch, src_table, tgt_table):
    pe = jnp.asarray(_PE_PACKED)
    src_ids = src_token_ids_batch.astype(jnp.int32)
    tgt_ids = tgt_token_ids_batch.astype(jnp.int32)
    return _sc_embed(src_ids, tgt_ids, src_table, tgt_table, pe)
